# baseline (speedup 1.0000x reference)
import jax
import jax.numpy as jnp
from jax import lax
from jax.experimental import pallas as pl
from jax.experimental.pallas import tpu as pltpu
from jax.experimental.pallas import tpu_sc as plsc

B = 16384
L = 32
NC = 2
NS = 16
NW = NC * NS
RPW = B // NW
IC = RPW * L
HC = IC // 2


def _body(x_hbm, w_hbm, pt_hbm, out_hbm,
          idx_v, gath_v, x_v, out_v, psem, sem_a, sem_b):
    wid = lax.axis_index("s") * NC + lax.axis_index("c")
    base = wid * RPW
    sems = [sem_a, sem_b]

    def load_idx_half(h):
        def fire(l, carry):
            pltpu.async_copy(
                pt_hbm.at[l, pl.ds(base, RPW)],
                idx_v.at[pl.ds(pl.multiple_of(l * RPW, 8), RPW)], psem)
            return carry

        lax.fori_loop(h * (L // 2), (h + 1) * (L // 2), fire, 0)
        pltpu.make_async_copy(
            pt_hbm.at[0, pl.ds(0, HC)], idx_v.at[pl.ds(h * HC, HC)],
            psem).wait()

    pltpu.sync_copy(x_hbm.at[pl.ds(base, RPW)], x_v)

    copies = []
    for h in range(2):
        load_idx_half(h)
        copies.append(pltpu.async_copy(
            w_hbm.at[idx_v.at[pl.ds(h * HC, HC)]],
            gath_v.at[pl.ds(h * HC, HC)], sems[h]))

    for h in range(2):
        copies[h].wait()

        def g_body(g, carry):
            row0 = pl.multiple_of(g * 16, 16)

            def l_body(l, acc):
                base4 = pl.multiple_of(h * HC + l * (4 * RPW), 8)
                for j in range(4):
                    acc = acc * gath_v[pl.ds(row0 + base4 + j * RPW, 16)]
                return acc

            init = x_v[pl.ds(row0, 16)] if h == 0 else out_v[pl.ds(row0, 16)]
            acc = lax.fori_loop(0, L // 8, l_body, init)
            out_v[pl.ds(row0, 16)] = acc
            return carry

        lax.fori_loop(0, RPW // 16, g_body, 0)

    pltpu.sync_copy(out_v, out_hbm.at[pl.ds(base, RPW)])


def kernel(x, weights, paths):
    paths_t = paths.astype(jnp.int32).T
    mesh = plsc.VectorSubcoreMesh(core_axis_name="c", subcore_axis_name="s")
    f = pl.kernel(
        _body,
        out_type=jax.ShapeDtypeStruct((B,), jnp.float32),
        mesh=mesh,
        scratch_types=[
            pltpu.VMEM((IC,), jnp.int32),
            pltpu.VMEM((IC,), jnp.float32),
            pltpu.VMEM((RPW,), jnp.float32),
            pltpu.VMEM((RPW,), jnp.float32),
            pltpu.SemaphoreType.DMA,
            pltpu.SemaphoreType.DMA,
            pltpu.SemaphoreType.DMA,
        ],
        compiler_params=pltpu.CompilerParams(needs_layout_passes=False),
    )
    return f(x, weights, paths_t)
